# initial kernel scaffold (unmeasured)
import functools

import jax
import jax.numpy as jnp
from jax import lax
from jax.experimental import pallas as pl
from jax.experimental.pallas import tpu as pltpu

N_DEV = 8
NQ = 4


def kernel(x, w_mat, scale_x, scale_w):
    M, _ = x.shape
    _, N = w_mat.shape
    m_per = M // N_DEV
    nq = N // NQ

    def body(x_ref, w_ref, sx_ref, sw_ref, out_ref,
             a_ref, b_ref, send_sem, recv_sem, credit_sem):
        my = lax.axis_index("i")
        left = lax.rem(my + N_DEV - 1, N_DEV)
        right = lax.rem(my + 1, N_DEV)

        def partial_q(c, q):
            xc = x_ref[pl.ds(c * m_per, m_per), :]
            wq = w_ref[:, q * nq:(q + 1) * nq]
            return lax.dot_general(
                xc, wq, (((1,), (0,)), ((), ())),
                preferred_element_type=jnp.float32)

        c0 = lax.rem(my + N_DEV - 1, N_DEV)
        for q in range(NQ):
            a_ref[:, q * nq:(q + 1) * nq] = (
                partial_q(c0, q).astype(jnp.bfloat16))

        barrier_sem = pltpu.get_barrier_semaphore()
        for nbr in (left, right):
            pl.semaphore_signal(barrier_sem, inc=1, device_id=(nbr,),
                                device_id_type=pl.DeviceIdType.MESH)
        pl.semaphore_wait(barrier_sem, 2)

        scale = sx_ref[0] * sw_ref[0]

        for h in range(N_DEV - 1):
            if h > 0:
                pl.semaphore_wait(credit_sem, 1)
            rdma = pltpu.make_async_remote_copy(
                src_ref=a_ref, dst_ref=b_ref,
                send_sem=send_sem, recv_sem=recv_sem,
                device_id=(right,), device_id_type=pl.DeviceIdType.MESH)
            rdma.start()
            rdma.wait()

            c = lax.rem(my + (2 * N_DEV - 2 - h), N_DEV)
            if h < N_DEV - 2:
                for q in range(NQ):
                    a_ref[:, q * nq:(q + 1) * nq] = (
                        b_ref[:, q * nq:(q + 1) * nq].astype(jnp.float32)
                        + partial_q(c, q)).astype(jnp.bfloat16)
                pl.semaphore_signal(credit_sem, inc=1, device_id=(left,),
                                    device_id_type=pl.DeviceIdType.MESH)
            else:
                for q in range(NQ):
                    out_ref[:, q * nq:(q + 1) * nq] = (
                        b_ref[:, q * nq:(q + 1) * nq].astype(jnp.float32)
                        + partial_q(c, q)) * scale

        @functools.partial(pl.run_scoped,
                           exit_sem=pltpu.SemaphoreType.REGULAR)
        def _(exit_sem):
            for nbr in (left, right):
                pl.semaphore_signal(exit_sem, inc=1, device_id=(nbr,),
                                    device_id_type=pl.DeviceIdType.MESH)
            pl.semaphore_wait(exit_sem, 2)

    return pl.pallas_call(
        body,
        out_shape=jax.ShapeDtypeStruct((m_per, N), jnp.float32),
        in_specs=[
            pl.BlockSpec(memory_space=pltpu.VMEM),
            pl.BlockSpec(memory_space=pltpu.VMEM),
            pl.BlockSpec(memory_space=pltpu.SMEM),
            pl.BlockSpec(memory_space=pltpu.SMEM),
        ],
        out_specs=pl.BlockSpec(memory_space=pltpu.VMEM),
        scratch_shapes=[
            pltpu.VMEM((m_per, N), jnp.bfloat16),
            pltpu.VMEM((m_per, N), jnp.bfloat16),
            pltpu.SemaphoreType.DMA,
            pltpu.SemaphoreType.DMA,
            pltpu.SemaphoreType.REGULAR,
        ],
        compiler_params=pltpu.CompilerParams(collective_id=0),
    )(x, w_mat, scale_x, scale_w)


# baseline (device time: 714705 ns/iter reference)
import functools

import jax
import jax.numpy as jnp
from jax import lax
from jax.experimental import pallas as pl
from jax.experimental.pallas import tpu as pltpu

N_DEV = 8
NQ = 4


def kernel(x, w_mat, scale_x, scale_w):
    M, _ = x.shape
    _, N = w_mat.shape
    m_per = M // N_DEV
    nq = N // NQ

    x = x.astype(jnp.float8_e5m2)
    w_mat = w_mat.astype(jnp.float8_e5m2)

    def body(x_ref, w_ref, sx_ref, sw_ref, out_ref,
             a_ref, b_ref, send_sem, recv_sem, credit_sem):
        my = lax.axis_index("i")
        left = lax.rem(my + N_DEV - 1, N_DEV)
        right = lax.rem(my + 1, N_DEV)

        def partial_q(c, q):
            xc = x_ref[pl.ds(c * m_per, m_per), :]
            wq = w_ref[:, q * nq:(q + 1) * nq]
            return lax.dot_general(
                xc, wq, (((1,), (0,)), ((), ())),
                preferred_element_type=jnp.float32)

        c0 = lax.rem(my + N_DEV - 1, N_DEV)
        for q in range(NQ):
            a_ref[:, q * nq:(q + 1) * nq] = (
                partial_q(c0, q).astype(jnp.bfloat16))

        barrier_sem = pltpu.get_barrier_semaphore()
        for nbr in (left, right):
            pl.semaphore_signal(barrier_sem, inc=1, device_id=(nbr,),
                                device_id_type=pl.DeviceIdType.MESH)
        pl.semaphore_wait(barrier_sem, 2)

        scale = sx_ref[0] * sw_ref[0]

        for h in range(N_DEV - 1):
            if h > 0:
                pl.semaphore_wait(credit_sem, 1)
            rdma = pltpu.make_async_remote_copy(
                src_ref=a_ref, dst_ref=b_ref,
                send_sem=send_sem, recv_sem=recv_sem,
                device_id=(right,), device_id_type=pl.DeviceIdType.MESH)
            rdma.start()
            rdma.wait()

            c = lax.rem(my + (2 * N_DEV - 2 - h), N_DEV)
            if h < N_DEV - 2:
                for q in range(NQ):
                    a_ref[:, q * nq:(q + 1) * nq] = (
                        b_ref[:, q * nq:(q + 1) * nq].astype(jnp.float32)
                        + partial_q(c, q)).astype(jnp.bfloat16)
                pl.semaphore_signal(credit_sem, inc=1, device_id=(left,),
                                    device_id_type=pl.DeviceIdType.MESH)
            else:
                for q in range(NQ):
                    out_ref[:, q * nq:(q + 1) * nq] = (
                        b_ref[:, q * nq:(q + 1) * nq].astype(jnp.float32)
                        + partial_q(c, q)) * scale

        @functools.partial(pl.run_scoped,
                           exit_sem=pltpu.SemaphoreType.REGULAR)
        def _(exit_sem):
            for nbr in (left, right):
                pl.semaphore_signal(exit_sem, inc=1, device_id=(nbr,),
                                    device_id_type=pl.DeviceIdType.MESH)
            pl.semaphore_wait(exit_sem, 2)

    return pl.pallas_call(
        body,
        out_shape=jax.ShapeDtypeStruct((m_per, N), jnp.float32),
        in_specs=[
            pl.BlockSpec(memory_space=pltpu.VMEM),
            pl.BlockSpec(memory_space=pltpu.VMEM),
            pl.BlockSpec(memory_space=pltpu.SMEM),
            pl.BlockSpec(memory_space=pltpu.SMEM),
        ],
        out_specs=pl.BlockSpec(memory_space=pltpu.VMEM),
        scratch_shapes=[
            pltpu.VMEM((m_per, N), jnp.bfloat16),
            pltpu.VMEM((m_per, N), jnp.bfloat16),
            pltpu.SemaphoreType.DMA,
            pltpu.SemaphoreType.DMA,
            pltpu.SemaphoreType.REGULAR,
        ],
        compiler_params=pltpu.CompilerParams(
            collective_id=0, vmem_limit_bytes=64 * 1024 * 1024),
    )(x, w_mat, scale_x, scale_w)


# device time: 388581 ns/iter; 1.8393x vs baseline; 1.8393x over previous
import functools

import jax
import jax.numpy as jnp
from jax import lax
from jax.experimental import pallas as pl
from jax.experimental.pallas import tpu as pltpu

N_DEV = 8
NQ = 4


def kernel(x, w_mat, scale_x, scale_w):
    M, _ = x.shape
    _, N = w_mat.shape
    m_per = M // N_DEV
    nq = N // NQ

    x = x.astype(jnp.float8_e5m2)
    w_mat = w_mat.astype(jnp.float8_e5m2)

    def body(x_ref, w_ref, sx_ref, sw_ref, out_ref,
             xr_ref, wbuf_ref,
             a2a_send_sems, a2a_recv_sems, ssem, rsem, credit_sem):
        my = lax.axis_index("i")
        left = lax.rem(my + N_DEV - 1, N_DEV)
        right = lax.rem(my + 1, N_DEV)
        mesh = pl.DeviceIdType.MESH

        barrier_sem = pltpu.get_barrier_semaphore()
        for d in range(1, N_DEV):
            peer = lax.rem(my + d, N_DEV)
            pl.semaphore_signal(barrier_sem, inc=1, device_id=(peer,),
                                device_id_type=mesh)
        pl.semaphore_wait(barrier_sem, N_DEV - 1)

        a2a = []
        for d in range(1, N_DEV):
            t = lax.rem(my + d, N_DEV)
            desc = pltpu.make_async_remote_copy(
                src_ref=x_ref.at[pl.ds(t * m_per, m_per), :],
                dst_ref=xr_ref.at[:, pl.ds(my * m_per, m_per)],
                send_sem=a2a_send_sems.at[d],
                recv_sem=a2a_recv_sems.at[d],
                device_id=(t,), device_id_type=mesh)
            desc.start()
            a2a.append(desc)

        for d in range(1, N_DEV):
            j = lax.rem(my + N_DEV - d, N_DEV)
            recv = pltpu.make_async_remote_copy(
                src_ref=x_ref.at[pl.ds(0, m_per), :],
                dst_ref=xr_ref.at[:, pl.ds(j * m_per, m_per)],
                send_sem=a2a_send_sems.at[d],
                recv_sem=a2a_recv_sems.at[d],
                device_id=(j,), device_id_type=mesh)
            recv.wait_recv()

        scale = sx_ref[0] * sw_ref[0]

        for h in range(N_DEV - 1):
            if h >= 2:
                pl.semaphore_wait(credit_sem, 1)
            src = w_ref if h == 0 else wbuf_ref.at[(h - 1) % 2]
            rdma = pltpu.make_async_remote_copy(
                src_ref=src, dst_ref=wbuf_ref.at[h % 2],
                send_sem=ssem.at[h % 2], recv_sem=rsem.at[h % 2],
                device_id=(right,), device_id_type=mesh)
            rdma.start()

            if h == 0:
                for q in range(NQ):
                    out_ref[:, q * nq:(q + 1) * nq] = lax.dot_general(
                        x_ref[pl.ds(my * m_per, m_per), :],
                        w_ref[:, q * nq:(q + 1) * nq],
                        (((1,), (0,)), ((), ())),
                        preferred_element_type=jnp.float32)
            else:
                jcol = lax.rem(my + N_DEV - h, N_DEV) * m_per
                for q in range(NQ):
                    out_ref[:, q * nq:(q + 1) * nq] = (
                        out_ref[:, q * nq:(q + 1) * nq]
                        + lax.dot_general(
                            xr_ref[:, pl.ds(jcol, m_per)],
                            wbuf_ref[(h - 1) % 2, :, q * nq:(q + 1) * nq],
                            (((1,), (0,)), ((), ())),
                            preferred_element_type=jnp.float32))

            rdma.wait_send()
            if 1 <= h <= 5:
                pl.semaphore_signal(credit_sem, inc=1, device_id=(left,),
                                    device_id_type=mesh)
            rdma.wait_recv()

        jcol = lax.rem(my + 1, N_DEV) * m_per
        for q in range(NQ):
            out_ref[:, q * nq:(q + 1) * nq] = (
                out_ref[:, q * nq:(q + 1) * nq]
                + lax.dot_general(
                    xr_ref[:, pl.ds(jcol, m_per)],
                    wbuf_ref[0, :, q * nq:(q + 1) * nq],
                    (((1,), (0,)), ((), ())),
                    preferred_element_type=jnp.float32)) * scale

        for desc in a2a:
            desc.wait_send()

        @functools.partial(pl.run_scoped,
                           exit_sem=pltpu.SemaphoreType.REGULAR)
        def _(exit_sem):
            for d in range(1, N_DEV):
                peer = lax.rem(my + d, N_DEV)
                pl.semaphore_signal(exit_sem, inc=1, device_id=(peer,),
                                    device_id_type=mesh)
            pl.semaphore_wait(exit_sem, N_DEV - 1)

    return pl.pallas_call(
        body,
        out_shape=jax.ShapeDtypeStruct((m_per, N), jnp.float32),
        in_specs=[
            pl.BlockSpec(memory_space=pltpu.VMEM),
            pl.BlockSpec(memory_space=pltpu.VMEM),
            pl.BlockSpec(memory_space=pltpu.SMEM),
            pl.BlockSpec(memory_space=pltpu.SMEM),
        ],
        out_specs=pl.BlockSpec(memory_space=pltpu.VMEM),
        scratch_shapes=[
            pltpu.VMEM((m_per, M), jnp.float8_e5m2),
            pltpu.VMEM((2, m_per, N), jnp.float8_e5m2),
            pltpu.SemaphoreType.DMA((N_DEV,)),
            pltpu.SemaphoreType.DMA((N_DEV,)),
            pltpu.SemaphoreType.DMA((2,)),
            pltpu.SemaphoreType.DMA((2,)),
            pltpu.SemaphoreType.REGULAR,
        ],
        compiler_params=pltpu.CompilerParams(
            collective_id=0, vmem_limit_bytes=64 * 1024 * 1024),
    )(x, w_mat, scale_x, scale_w)


# device time: 233075 ns/iter; 3.0664x vs baseline; 1.6672x over previous
import functools

import jax
import jax.numpy as jnp
from jax import lax
from jax.experimental import pallas as pl
from jax.experimental.pallas import tpu as pltpu

N_DEV = 8
NH = 2


def kernel(x, w_mat, scale_x, scale_w):
    M, _ = x.shape
    _, N = w_mat.shape
    m_per = M // N_DEV
    half = N // 2
    nh = half // NH

    x = x.astype(jnp.float8_e5m2)
    w_mat = w_mat.astype(jnp.float8_e5m2)

    def body(x_ref, w_ref, sx_ref, sw_ref, out_ref,
             xr_ref, lbuf_ref, rbuf_ref,
             a2a_send_sems, a2a_recv_sems,
             cw_ssem, cw_rsem, ccw_ssem, ccw_rsem,
             cw_credit, ccw_credit):
        my = lax.axis_index("i")
        left = lax.rem(my + N_DEV - 1, N_DEV)
        right = lax.rem(my + 1, N_DEV)
        mesh = pl.DeviceIdType.MESH

        barrier_sem = pltpu.get_barrier_semaphore()
        for d in range(1, N_DEV):
            peer = lax.rem(my + d, N_DEV)
            pl.semaphore_signal(barrier_sem, inc=1, device_id=(peer,),
                                device_id_type=mesh)
        pl.semaphore_wait(barrier_sem, N_DEV - 1)

        a2a = []
        for d in range(1, N_DEV):
            t = lax.rem(my + d, N_DEV)
            desc = pltpu.make_async_remote_copy(
                src_ref=x_ref.at[pl.ds(t * m_per, m_per), :],
                dst_ref=xr_ref.at[:, pl.ds(my * m_per, m_per)],
                send_sem=a2a_send_sems.at[d],
                recv_sem=a2a_recv_sems.at[d],
                device_id=(t,), device_id_type=mesh)
            desc.start()
            a2a.append(desc)
        xr_ref[:, pl.ds(my * m_per, m_per)] = x_ref[pl.ds(my * m_per, m_per), :]
        for d in range(1, N_DEV):
            j = lax.rem(my + N_DEV - d, N_DEV)
            recv = pltpu.make_async_remote_copy(
                src_ref=x_ref.at[pl.ds(0, m_per), :],
                dst_ref=xr_ref.at[:, pl.ds(j * m_per, m_per)],
                send_sem=a2a_send_sems.at[d],
                recv_sem=a2a_recv_sems.at[d],
                device_id=(j,), device_id_type=mesh)
            recv.wait_recv()

        scale = sx_ref[0] * sw_ref[0]

        def gemm(jcol, w_half_ref, col0, accumulate, scaled=False):
            for q in range(NH):
                acc = lax.dot_general(
                    xr_ref[:, pl.ds(jcol, m_per)],
                    w_half_ref[:, q * nh:(q + 1) * nh],
                    (((1,), (0,)), ((), ())),
                    preferred_element_type=jnp.float32)
                sl = slice(col0 + q * nh, col0 + (q + 1) * nh)
                if accumulate:
                    acc = out_ref[:, sl] + acc
                if scaled:
                    acc = acc * scale
                out_ref[:, sl] = acc

        for h in range(N_DEV - 1):
            if h >= 2:
                pl.semaphore_wait(cw_credit, 1)
                pl.semaphore_wait(ccw_credit, 1)
            cw_src = (w_ref.at[:, pl.ds(0, half)] if h == 0
                      else lbuf_ref.at[(h - 1) % 2])
            cw = pltpu.make_async_remote_copy(
                src_ref=cw_src, dst_ref=lbuf_ref.at[h % 2],
                send_sem=cw_ssem.at[h % 2], recv_sem=cw_rsem.at[h % 2],
                device_id=(right,), device_id_type=mesh)
            cw.start()
            ccw_src = (w_ref.at[:, pl.ds(half, half)] if h == 0
                       else rbuf_ref.at[(h - 1) % 2])
            ccw = pltpu.make_async_remote_copy(
                src_ref=ccw_src, dst_ref=rbuf_ref.at[h % 2],
                send_sem=ccw_ssem.at[h % 2], recv_sem=ccw_rsem.at[h % 2],
                device_id=(left,), device_id_type=mesh)
            ccw.start()

            if h == 0:
                gemm(my * m_per, w_ref.at[:, pl.ds(0, half)], 0,
                     accumulate=False)
                gemm(my * m_per, w_ref.at[:, pl.ds(half, half)], half,
                     accumulate=False)
            else:
                gemm(lax.rem(my + N_DEV - h, N_DEV) * m_per,
                     lbuf_ref.at[(h - 1) % 2], 0, accumulate=True)
                gemm(lax.rem(my + h, N_DEV) * m_per,
                     rbuf_ref.at[(h - 1) % 2], half, accumulate=True)

            cw.wait_send()
            ccw.wait_send()
            if 1 <= h <= 5:
                pl.semaphore_signal(cw_credit, inc=1, device_id=(left,),
                                    device_id_type=mesh)
                pl.semaphore_signal(ccw_credit, inc=1, device_id=(right,),
                                    device_id_type=mesh)
            cw.wait_recv()
            ccw.wait_recv()

        gemm(lax.rem(my + 1, N_DEV) * m_per, lbuf_ref.at[0], 0,
             accumulate=True, scaled=True)
        gemm(lax.rem(my + N_DEV - 1, N_DEV) * m_per, rbuf_ref.at[0], half,
             accumulate=True, scaled=True)

        for desc in a2a:
            desc.wait_send()

        @functools.partial(pl.run_scoped,
                           exit_sem=pltpu.SemaphoreType.REGULAR)
        def _(exit_sem):
            for d in range(1, N_DEV):
                peer = lax.rem(my + d, N_DEV)
                pl.semaphore_signal(exit_sem, inc=1, device_id=(peer,),
                                    device_id_type=mesh)
            pl.semaphore_wait(exit_sem, N_DEV - 1)

    return pl.pallas_call(
        body,
        out_shape=jax.ShapeDtypeStruct((m_per, N), jnp.float32),
        in_specs=[
            pl.BlockSpec(memory_space=pltpu.VMEM),
            pl.BlockSpec(memory_space=pltpu.VMEM),
            pl.BlockSpec(memory_space=pltpu.SMEM),
            pl.BlockSpec(memory_space=pltpu.SMEM),
        ],
        out_specs=pl.BlockSpec(memory_space=pltpu.VMEM),
        scratch_shapes=[
            pltpu.VMEM((m_per, M), jnp.float8_e5m2),
            pltpu.VMEM((2, m_per, half), jnp.float8_e5m2),
            pltpu.VMEM((2, m_per, half), jnp.float8_e5m2),
            pltpu.SemaphoreType.DMA((N_DEV,)),
            pltpu.SemaphoreType.DMA((N_DEV,)),
            pltpu.SemaphoreType.DMA((2,)),
            pltpu.SemaphoreType.DMA((2,)),
            pltpu.SemaphoreType.DMA((2,)),
            pltpu.SemaphoreType.DMA((2,)),
            pltpu.SemaphoreType.REGULAR,
            pltpu.SemaphoreType.REGULAR,
        ],
        compiler_params=pltpu.CompilerParams(
            collective_id=0, vmem_limit_bytes=64 * 1024 * 1024),
    )(x, w_mat, scale_x, scale_w)


# device time: 231141 ns/iter; 3.0921x vs baseline; 1.0084x over previous
import functools

import jax
import jax.numpy as jnp
from jax import lax
from jax.experimental import pallas as pl
from jax.experimental.pallas import tpu as pltpu

N_DEV = 8
NH = 2


def kernel(x, w_mat, scale_x, scale_w):
    M, _ = x.shape
    _, N = w_mat.shape
    m_per = M // N_DEV
    half = N // 2
    nh = half // NH

    x = x.astype(jnp.float8_e5m2)
    w_mat = w_mat.astype(jnp.float8_e5m2)

    def body(x_ref, w_ref, sx_ref, sw_ref, out_ref,
             xr_ref, lbuf_ref, rbuf_ref,
             a2a_send_sems, a2a_recv_sems,
             cw_ssem, cw_rsem, ccw_ssem, ccw_rsem,
             cw_credit, ccw_credit):
        my = lax.axis_index("i")
        left = lax.rem(my + N_DEV - 1, N_DEV)
        right = lax.rem(my + 1, N_DEV)
        mesh = pl.DeviceIdType.MESH

        barrier_sem = pltpu.get_barrier_semaphore()
        for d in range(1, N_DEV):
            peer = lax.rem(my + d, N_DEV)
            pl.semaphore_signal(barrier_sem, inc=1, device_id=(peer,),
                                device_id_type=mesh)
        pl.semaphore_wait(barrier_sem, N_DEV - 1)

        a2a = []
        for d in range(1, N_DEV):
            t = lax.rem(my + d, N_DEV)
            desc = pltpu.make_async_remote_copy(
                src_ref=x_ref.at[pl.ds(t * m_per, m_per), :],
                dst_ref=xr_ref.at[:, pl.ds(my * m_per, m_per)],
                send_sem=a2a_send_sems.at[d],
                recv_sem=a2a_recv_sems.at[d],
                device_id=(t,), device_id_type=mesh)
            desc.start()
            a2a.append(desc)
        xr_ref[:, pl.ds(my * m_per, m_per)] = x_ref[pl.ds(my * m_per, m_per), :]

        scale = sx_ref[0] * sw_ref[0]

        def gemm(jcol, w_half_ref, col0, accumulate, scaled=False):
            for q in range(NH):
                acc = lax.dot_general(
                    xr_ref[:, pl.ds(jcol, m_per)],
                    w_half_ref[:, q * nh:(q + 1) * nh],
                    (((1,), (0,)), ((), ())),
                    preferred_element_type=jnp.float32)
                sl = slice(col0 + q * nh, col0 + (q + 1) * nh)
                if accumulate:
                    acc = out_ref[:, sl] + acc
                if scaled:
                    acc = acc * scale
                out_ref[:, sl] = acc

        cw0 = pltpu.make_async_remote_copy(
            src_ref=w_ref.at[:, pl.ds(0, half)], dst_ref=lbuf_ref.at[0],
            send_sem=cw_ssem.at[0], recv_sem=cw_rsem.at[0],
            device_id=(right,), device_id_type=mesh)
        cw0.start()
        ccw0 = pltpu.make_async_remote_copy(
            src_ref=w_ref.at[:, pl.ds(half, half)], dst_ref=rbuf_ref.at[0],
            send_sem=ccw_ssem.at[0], recv_sem=ccw_rsem.at[0],
            device_id=(left,), device_id_type=mesh)
        ccw0.start()
        gemm(my * m_per, w_ref.at[:, pl.ds(0, half)], 0, accumulate=False)
        gemm(my * m_per, w_ref.at[:, pl.ds(half, half)], half,
             accumulate=False)

        for d in range(1, N_DEV):
            j = lax.rem(my + N_DEV - d, N_DEV)
            recv = pltpu.make_async_remote_copy(
                src_ref=x_ref.at[pl.ds(0, m_per), :],
                dst_ref=xr_ref.at[:, pl.ds(j * m_per, m_per)],
                send_sem=a2a_send_sems.at[d],
                recv_sem=a2a_recv_sems.at[d],
                device_id=(j,), device_id_type=mesh)
            recv.wait_recv()

        cw0.wait_send()
        ccw0.wait_send()
        cw0.wait_recv()
        ccw0.wait_recv()

        for h in range(1, N_DEV - 1):
            if h >= 2:
                pl.semaphore_wait(cw_credit, 1)
                pl.semaphore_wait(ccw_credit, 1)
            cw = pltpu.make_async_remote_copy(
                src_ref=lbuf_ref.at[(h - 1) % 2], dst_ref=lbuf_ref.at[h % 2],
                send_sem=cw_ssem.at[h % 2], recv_sem=cw_rsem.at[h % 2],
                device_id=(right,), device_id_type=mesh)
            cw.start()
            ccw = pltpu.make_async_remote_copy(
                src_ref=rbuf_ref.at[(h - 1) % 2], dst_ref=rbuf_ref.at[h % 2],
                send_sem=ccw_ssem.at[h % 2], recv_sem=ccw_rsem.at[h % 2],
                device_id=(left,), device_id_type=mesh)
            ccw.start()

            gemm(lax.rem(my + N_DEV - h, N_DEV) * m_per,
                 lbuf_ref.at[(h - 1) % 2], 0, accumulate=True)
            gemm(lax.rem(my + h, N_DEV) * m_per,
                 rbuf_ref.at[(h - 1) % 2], half, accumulate=True)

            cw.wait_send()
            ccw.wait_send()
            if 1 <= h <= 5:
                pl.semaphore_signal(cw_credit, inc=1, device_id=(left,),
                                    device_id_type=mesh)
                pl.semaphore_signal(ccw_credit, inc=1, device_id=(right,),
                                    device_id_type=mesh)
            cw.wait_recv()
            ccw.wait_recv()

        gemm(lax.rem(my + 1, N_DEV) * m_per, lbuf_ref.at[0], 0,
             accumulate=True, scaled=True)
        gemm(lax.rem(my + N_DEV - 1, N_DEV) * m_per, rbuf_ref.at[0], half,
             accumulate=True, scaled=True)

        for desc in a2a:
            desc.wait_send()

        @functools.partial(pl.run_scoped,
                           exit_sem=pltpu.SemaphoreType.REGULAR)
        def _(exit_sem):
            for d in range(1, N_DEV):
                peer = lax.rem(my + d, N_DEV)
                pl.semaphore_signal(exit_sem, inc=1, device_id=(peer,),
                                    device_id_type=mesh)
            pl.semaphore_wait(exit_sem, N_DEV - 1)

    return pl.pallas_call(
        body,
        out_shape=jax.ShapeDtypeStruct((m_per, N), jnp.float32),
        in_specs=[
            pl.BlockSpec(memory_space=pltpu.VMEM),
            pl.BlockSpec(memory_space=pltpu.VMEM),
            pl.BlockSpec(memory_space=pltpu.SMEM),
            pl.BlockSpec(memory_space=pltpu.SMEM),
        ],
        out_specs=pl.BlockSpec(memory_space=pltpu.VMEM),
        scratch_shapes=[
            pltpu.VMEM((m_per, M), jnp.float8_e5m2),
            pltpu.VMEM((2, m_per, half), jnp.float8_e5m2),
            pltpu.VMEM((2, m_per, half), jnp.float8_e5m2),
            pltpu.SemaphoreType.DMA((N_DEV,)),
            pltpu.SemaphoreType.DMA((N_DEV,)),
            pltpu.SemaphoreType.DMA((2,)),
            pltpu.SemaphoreType.DMA((2,)),
            pltpu.SemaphoreType.DMA((2,)),
            pltpu.SemaphoreType.DMA((2,)),
            pltpu.SemaphoreType.REGULAR,
            pltpu.SemaphoreType.REGULAR,
        ],
        compiler_params=pltpu.CompilerParams(
            collective_id=0, vmem_limit_bytes=64 * 1024 * 1024),
    )(x, w_mat, scale_x, scale_w)


# device time: 231072 ns/iter; 3.0930x vs baseline; 1.0003x over previous
import functools

import jax
import jax.numpy as jnp
from jax import lax
from jax.experimental import pallas as pl
from jax.experimental.pallas import tpu as pltpu

N_DEV = 8
NH = 2


def kernel(x, w_mat, scale_x, scale_w):
    M, _ = x.shape
    _, N = w_mat.shape
    m_per = M // N_DEV
    half = N // 2
    nh = half // NH

    x = x.astype(jnp.float8_e5m2)
    w_mat = w_mat.astype(jnp.float8_e5m2)

    def body(x_ref, w_ref, sx_ref, sw_ref, out_ref,
             xr_ref, lbuf_ref, rbuf_ref,
             a2a_send_sems, a2a_recv_sems,
             cw_ssem, cw_rsem, ccw_ssem, ccw_rsem,
             cw_credit, ccw_credit):
        my = lax.axis_index("i")
        left = lax.rem(my + N_DEV - 1, N_DEV)
        right = lax.rem(my + 1, N_DEV)
        mesh = pl.DeviceIdType.MESH

        barrier_sem = pltpu.get_barrier_semaphore()
        for d in range(1, N_DEV):
            peer = lax.rem(my + d, N_DEV)
            pl.semaphore_signal(barrier_sem, inc=1, device_id=(peer,),
                                device_id_type=mesh)
        pl.semaphore_wait(barrier_sem, N_DEV - 1)

        a2a = []
        for d in range(1, N_DEV):
            t = lax.rem(my + d, N_DEV)
            desc = pltpu.make_async_remote_copy(
                src_ref=x_ref.at[pl.ds(t * m_per, m_per), :],
                dst_ref=xr_ref.at[:, pl.ds(my * m_per, m_per)],
                send_sem=a2a_send_sems.at[d],
                recv_sem=a2a_recv_sems.at[d],
                device_id=(t,), device_id_type=mesh)
            desc.start()
            a2a.append(desc)
        xr_ref[:, pl.ds(my * m_per, m_per)] = x_ref[pl.ds(my * m_per, m_per), :]

        scale = sx_ref[0] * sw_ref[0]

        def gemm(jcol, w_half_ref, col0, accumulate, scaled=False):
            for q in range(NH):
                acc = lax.dot_general(
                    xr_ref[:, pl.ds(jcol, m_per)],
                    w_half_ref[:, q * nh:(q + 1) * nh],
                    (((1,), (0,)), ((), ())),
                    preferred_element_type=jnp.float32)
                sl = slice(col0 + q * nh, col0 + (q + 1) * nh)
                if accumulate:
                    acc = out_ref[:, sl] + acc
                if scaled:
                    acc = acc * scale
                out_ref[:, sl] = acc

        cw0 = pltpu.make_async_remote_copy(
            src_ref=w_ref.at[:, pl.ds(0, half)], dst_ref=lbuf_ref.at[0],
            send_sem=cw_ssem.at[0], recv_sem=cw_rsem.at[0],
            device_id=(right,), device_id_type=mesh)
        cw0.start()
        ccw0 = pltpu.make_async_remote_copy(
            src_ref=w_ref.at[:, pl.ds(half, half)], dst_ref=rbuf_ref.at[0],
            send_sem=ccw_ssem.at[0], recv_sem=ccw_rsem.at[0],
            device_id=(left,), device_id_type=mesh)
        ccw0.start()
        gemm(my * m_per, w_ref.at[:, pl.ds(0, half)], 0, accumulate=False)
        gemm(my * m_per, w_ref.at[:, pl.ds(half, half)], half,
             accumulate=False)

        for d in range(1, N_DEV):
            j = lax.rem(my + N_DEV - d, N_DEV)
            recv = pltpu.make_async_remote_copy(
                src_ref=x_ref.at[pl.ds(0, m_per), :],
                dst_ref=xr_ref.at[:, pl.ds(j * m_per, m_per)],
                send_sem=a2a_send_sems.at[d],
                recv_sem=a2a_recv_sems.at[d],
                device_id=(j,), device_id_type=mesh)
            recv.wait_recv()

        cw0.wait_recv()
        ccw0.wait_recv()
        cw0.wait_send()
        ccw0.wait_send()

        for h in range(1, N_DEV - 1):
            if h >= 3:
                pl.semaphore_wait(cw_credit, 1)
                pl.semaphore_wait(ccw_credit, 1)
            cw = pltpu.make_async_remote_copy(
                src_ref=lbuf_ref.at[(h - 1) % 3], dst_ref=lbuf_ref.at[h % 3],
                send_sem=cw_ssem.at[h % 3], recv_sem=cw_rsem.at[h % 3],
                device_id=(right,), device_id_type=mesh)
            cw.start()
            ccw = pltpu.make_async_remote_copy(
                src_ref=rbuf_ref.at[(h - 1) % 3], dst_ref=rbuf_ref.at[h % 3],
                send_sem=ccw_ssem.at[h % 3], recv_sem=ccw_rsem.at[h % 3],
                device_id=(left,), device_id_type=mesh)
            ccw.start()

            gemm(lax.rem(my + N_DEV - h, N_DEV) * m_per,
                 lbuf_ref.at[(h - 1) % 3], 0, accumulate=True)
            gemm(lax.rem(my + h, N_DEV) * m_per,
                 rbuf_ref.at[(h - 1) % 3], half, accumulate=True)

            cw.wait_recv()
            ccw.wait_recv()
            cw.wait_send()
            ccw.wait_send()
            if 1 <= h <= 4:
                pl.semaphore_signal(cw_credit, inc=1, device_id=(left,),
                                    device_id_type=mesh)
                pl.semaphore_signal(ccw_credit, inc=1, device_id=(right,),
                                    device_id_type=mesh)

        gemm(lax.rem(my + 1, N_DEV) * m_per, lbuf_ref.at[0], 0,
             accumulate=True, scaled=True)
        gemm(lax.rem(my + N_DEV - 1, N_DEV) * m_per, rbuf_ref.at[0], half,
             accumulate=True, scaled=True)

        for desc in a2a:
            desc.wait_send()

        @functools.partial(pl.run_scoped,
                           exit_sem=pltpu.SemaphoreType.REGULAR)
        def _(exit_sem):
            for d in range(1, N_DEV):
                peer = lax.rem(my + d, N_DEV)
                pl.semaphore_signal(exit_sem, inc=1, device_id=(peer,),
                                    device_id_type=mesh)
            pl.semaphore_wait(exit_sem, N_DEV - 1)

    return pl.pallas_call(
        body,
        out_shape=jax.ShapeDtypeStruct((m_per, N), jnp.float32),
        in_specs=[
            pl.BlockSpec(memory_space=pltpu.VMEM),
            pl.BlockSpec(memory_space=pltpu.VMEM),
            pl.BlockSpec(memory_space=pltpu.SMEM),
            pl.BlockSpec(memory_space=pltpu.SMEM),
        ],
        out_specs=pl.BlockSpec(memory_space=pltpu.VMEM),
        scratch_shapes=[
            pltpu.VMEM((m_per, M), jnp.float8_e5m2),
            pltpu.VMEM((3, m_per, half), jnp.float8_e5m2),
            pltpu.VMEM((3, m_per, half), jnp.float8_e5m2),
            pltpu.SemaphoreType.DMA((N_DEV,)),
            pltpu.SemaphoreType.DMA((N_DEV,)),
            pltpu.SemaphoreType.DMA((3,)),
            pltpu.SemaphoreType.DMA((3,)),
            pltpu.SemaphoreType.DMA((3,)),
            pltpu.SemaphoreType.DMA((3,)),
            pltpu.SemaphoreType.REGULAR,
            pltpu.SemaphoreType.REGULAR,
        ],
        compiler_params=pltpu.CompilerParams(
            collective_id=0, vmem_limit_bytes=64 * 1024 * 1024),
    )(x, w_mat, scale_x, scale_w)


# device time: 181580 ns/iter; 3.9360x vs baseline; 1.2726x over previous
import functools

import jax
import jax.numpy as jnp
from jax import lax
from jax.experimental import pallas as pl
from jax.experimental.pallas import tpu as pltpu

N_DEV = 8
PART_COLS = (2816, 2688, 2688)
PART_OFF = (0, 2816, 5504)
ORDERS = ((1, 3, 4), (3, 4, 1), (4, 1, 3))


def _g(r, masks):
    v = 0
    for j in range(len(masks)):
        if r & (1 << j):
            v ^= masks[j]
    return v


def kernel(x, w_mat, scale_x, scale_w):
    M, _ = x.shape
    _, N = w_mat.shape
    m_per = M // N_DEV

    x = x.astype(jnp.float8_e5m2)
    w_mat = w_mat.astype(jnp.float8_e5m2)

    def body(x_ref, w_ref, sx_ref, sw_ref, out_ref, xr_ref,
             b0_ref, b1_ref, b2_ref,
             a2a_send_sems, a2a_recv_sems, ag_ssem, ag_rsem):
        my = lax.axis_index("i")
        mesh = pl.DeviceIdType.MESH
        bufs = (b0_ref, b1_ref, b2_ref)

        barrier_sem = pltpu.get_barrier_semaphore()
        for d in range(1, N_DEV):
            peer = lax.rem(my + d, N_DEV)
            pl.semaphore_signal(barrier_sem, inc=1, device_id=(peer,),
                                device_id_type=mesh)
        pl.semaphore_wait(barrier_sem, N_DEV - 1)

        a2a = []
        for d in range(1, N_DEV):
            t = lax.rem(my + d, N_DEV)
            desc = pltpu.make_async_remote_copy(
                src_ref=x_ref.at[pl.ds(t * m_per, m_per), :],
                dst_ref=xr_ref.at[:, pl.ds(my * m_per, m_per)],
                send_sem=a2a_send_sems.at[d],
                recv_sem=a2a_recv_sems.at[d],
                device_id=(t,), device_id_type=mesh)
            desc.start()
            a2a.append(desc)
        xr_ref[:, pl.ds(my * m_per, m_per)] = x_ref[pl.ds(my * m_per, m_per), :]

        scale = sx_ref[0] * sw_ref[0]

        def src_of(p, r):
            if r == 0:
                return w_ref.at[:, pl.ds(PART_OFF[p], PART_COLS[p])]
            s = r.bit_length() - 1
            return bufs[s].at[p].at[pl.ds((r - (1 << s)) * m_per, m_per),
                                    pl.ds(0, PART_COLS[p])]

        def gemm(p, origin_const, chunk_ref, accumulate):
            o = lax.bitwise_xor(my, origin_const) if origin_const else my
            xo = xr_ref[:, pl.ds(o * m_per, m_per)]
            cp = PART_COLS[p]
            for q in range(2):
                t0 = q * (cp // 2)
                tc = cp // 2
                acc = lax.dot_general(
                    xo, chunk_ref[:, t0:t0 + tc],
                    (((1,), (0,)), ((), ())),
                    preferred_element_type=jnp.float32)
                sl = slice(PART_OFF[p] + t0, PART_OFF[p] + t0 + tc)
                if accumulate:
                    acc = out_ref[:, sl] + acc
                out_ref[:, sl] = acc

        descs = {}
        for t in range(3):
            for p in range(3):
                mask = ORDERS[p][t]
                partner = lax.bitwise_xor(my, mask)
                for r in range(1 << t):
                    d = pltpu.make_async_remote_copy(
                        src_ref=src_of(p, r),
                        dst_ref=bufs[t].at[p].at[
                            pl.ds(r * m_per, m_per), pl.ds(0, PART_COLS[p])],
                        send_sem=ag_ssem.at[p, t],
                        recv_sem=ag_rsem.at[p, t],
                        device_id=(partner,), device_id_type=mesh)
                    d.start()
                    descs[(p, t, r)] = d

            if t == 0:
                for d in range(1, N_DEV):
                    j = lax.rem(my + N_DEV - d, N_DEV)
                    recv = pltpu.make_async_remote_copy(
                        src_ref=x_ref.at[pl.ds(0, m_per), :],
                        dst_ref=xr_ref.at[:, pl.ds(j * m_per, m_per)],
                        send_sem=a2a_send_sems.at[d],
                        recv_sem=a2a_recv_sems.at[d],
                        device_id=(j,), device_id_type=mesh)
                    recv.wait_recv()
                for p in range(3):
                    gemm(p, 0, w_ref.at[:, pl.ds(PART_OFF[p], PART_COLS[p])],
                         accumulate=False)
            else:
                for p in range(3):
                    mask_prev = ORDERS[p][t - 1]
                    for r in range(1 << (t - 1)):
                        oc = mask_prev ^ _g(r, ORDERS[p][:t - 1])
                        gemm(p, oc,
                             bufs[t - 1].at[p].at[pl.ds(r * m_per, m_per),
                                                  pl.ds(0, PART_COLS[p])],
                             accumulate=True)

            for p in range(3):
                for r in range(1 << t):
                    descs[(p, t, r)].wait_recv()
            for p in range(3):
                for r in range(1 << t):
                    descs[(p, t, r)].wait_send()

        for p in range(3):
            mask_prev = ORDERS[p][2]
            for r in range(4):
                oc = mask_prev ^ _g(r, ORDERS[p][:2])
                gemm(p, oc,
                     bufs[2].at[p].at[pl.ds(r * m_per, m_per),
                                      pl.ds(0, PART_COLS[p])],
                     accumulate=True)
        for q in range(4):
            sl = slice(q * (N // 4), (q + 1) * (N // 4))
            out_ref[:, sl] = out_ref[:, sl] * scale

        for desc in a2a:
            desc.wait_send()

        @functools.partial(pl.run_scoped,
                           exit_sem=pltpu.SemaphoreType.REGULAR)
        def _(exit_sem):
            for d in range(1, N_DEV):
                peer = lax.rem(my + d, N_DEV)
                pl.semaphore_signal(exit_sem, inc=1, device_id=(peer,),
                                    device_id_type=mesh)
            pl.semaphore_wait(exit_sem, N_DEV - 1)

    return pl.pallas_call(
        body,
        out_shape=jax.ShapeDtypeStruct((m_per, N), jnp.float32),
        in_specs=[
            pl.BlockSpec(memory_space=pltpu.VMEM),
            pl.BlockSpec(memory_space=pltpu.VMEM),
            pl.BlockSpec(memory_space=pltpu.SMEM),
            pl.BlockSpec(memory_space=pltpu.SMEM),
        ],
        out_specs=pl.BlockSpec(memory_space=pltpu.VMEM),
        scratch_shapes=[
            pltpu.VMEM((m_per, M), jnp.float8_e5m2),
            pltpu.VMEM((3, m_per, max(PART_COLS)), jnp.float8_e5m2),
            pltpu.VMEM((3, 2 * m_per, max(PART_COLS)), jnp.float8_e5m2),
            pltpu.VMEM((3, 4 * m_per, max(PART_COLS)), jnp.float8_e5m2),
            pltpu.SemaphoreType.DMA((N_DEV,)),
            pltpu.SemaphoreType.DMA((N_DEV,)),
            pltpu.SemaphoreType.DMA((3, 3)),
            pltpu.SemaphoreType.DMA((3, 3)),
        ],
        compiler_params=pltpu.CompilerParams(
            collective_id=0, vmem_limit_bytes=64 * 1024 * 1024),
    )(x, w_mat, scale_x, scale_w)


# device time: 176978 ns/iter; 4.0384x vs baseline; 1.0260x over previous
import functools

import jax
import jax.numpy as jnp
from jax import lax
from jax.experimental import pallas as pl
from jax.experimental.pallas import tpu as pltpu

N_DEV = 8
PART_COLS = (2816, 2688, 2688)
PART_OFF = (0, 2816, 5504)
ORDERS = ((1, 3, 4), (3, 4, 1), (4, 1, 3))


def _g(r, masks):
    v = 0
    for j in range(len(masks)):
        if r & (1 << j):
            v ^= masks[j]
    return v


def kernel(x, w_mat, scale_x, scale_w):
    M, _ = x.shape
    _, N = w_mat.shape
    m_per = M // N_DEV

    x = x.astype(jnp.float8_e5m2)
    w_mat = w_mat.astype(jnp.float8_e5m2)

    def body(x_ref, w_ref, sx_ref, sw_ref, out_ref, xr_ref,
             b0_ref, b1_ref, b2_ref,
             a2a_send_sems, a2a_recv_sems, ag_ssem, ag_rsem):
        my = lax.axis_index("i")
        mesh = pl.DeviceIdType.MESH
        bufs = (b0_ref, b1_ref, b2_ref)

        barrier_sem = pltpu.get_barrier_semaphore()
        for d in range(1, N_DEV):
            peer = lax.rem(my + d, N_DEV)
            pl.semaphore_signal(barrier_sem, inc=1, device_id=(peer,),
                                device_id_type=mesh)
        pl.semaphore_wait(barrier_sem, N_DEV - 1)

        a2a = []
        for d in range(1, N_DEV):
            t = lax.rem(my + d, N_DEV)
            desc = pltpu.make_async_remote_copy(
                src_ref=x_ref.at[pl.ds(t * m_per, m_per), :],
                dst_ref=xr_ref.at[:, pl.ds(my * m_per, m_per)],
                send_sem=a2a_send_sems.at[d],
                recv_sem=a2a_recv_sems.at[d],
                device_id=(t,), device_id_type=mesh)
            desc.start()
            a2a.append(desc)
        xr_ref[:, pl.ds(my * m_per, m_per)] = x_ref[pl.ds(my * m_per, m_per), :]

        scale = sx_ref[0] * sw_ref[0]

        def src_of(p, r):
            if r == 0:
                return w_ref.at[:, pl.ds(PART_OFF[p], PART_COLS[p])]
            s = r.bit_length() - 1
            return bufs[s].at[p].at[pl.ds((r - (1 << s)) * m_per, m_per),
                                    pl.ds(0, PART_COLS[p])]

        def gemm(p, origin_const, chunk_ref, accumulate, scaled=False):
            o = lax.bitwise_xor(my, origin_const) if origin_const else my
            xo = xr_ref[:, pl.ds(o * m_per, m_per)]
            cp = PART_COLS[p]
            for q in range(2):
                t0 = q * (cp // 2)
                tc = cp // 2
                acc = lax.dot_general(
                    xo, chunk_ref[:, t0:t0 + tc],
                    (((1,), (0,)), ((), ())),
                    preferred_element_type=jnp.float32)
                sl = slice(PART_OFF[p] + t0, PART_OFF[p] + t0 + tc)
                if accumulate:
                    acc = out_ref[:, sl] + acc
                if scaled:
                    acc = acc * scale
                out_ref[:, sl] = acc

        descs = {}
        for t in range(2):
            for p in range(3):
                mask = ORDERS[p][t]
                partner = lax.bitwise_xor(my, mask)
                for r in range(1 << t):
                    d = pltpu.make_async_remote_copy(
                        src_ref=src_of(p, r),
                        dst_ref=bufs[t].at[p].at[
                            pl.ds(r * m_per, m_per), pl.ds(0, PART_COLS[p])],
                        send_sem=ag_ssem.at[p, t],
                        recv_sem=ag_rsem.at[p, t],
                        device_id=(partner,), device_id_type=mesh)
                    d.start()
                    descs[(p, t, r)] = d

            if t == 0:
                for d in range(1, N_DEV):
                    j = lax.rem(my + N_DEV - d, N_DEV)
                    recv = pltpu.make_async_remote_copy(
                        src_ref=x_ref.at[pl.ds(0, m_per), :],
                        dst_ref=xr_ref.at[:, pl.ds(j * m_per, m_per)],
                        send_sem=a2a_send_sems.at[d],
                        recv_sem=a2a_recv_sems.at[d],
                        device_id=(j,), device_id_type=mesh)
                    recv.wait_recv()
                for p in range(3):
                    gemm(p, 0, w_ref.at[:, pl.ds(PART_OFF[p], PART_COLS[p])],
                         accumulate=False)
            else:
                for p in range(3):
                    mask_prev = ORDERS[p][t - 1]
                    for r in range(1 << (t - 1)):
                        oc = mask_prev ^ _g(r, ORDERS[p][:t - 1])
                        gemm(p, oc,
                             bufs[t - 1].at[p].at[pl.ds(r * m_per, m_per),
                                                  pl.ds(0, PART_COLS[p])],
                             accumulate=True)

            for p in range(3):
                for r in range(1 << t):
                    descs[(p, t, r)].wait_recv()
            for p in range(3):
                for r in range(1 << t):
                    descs[(p, t, r)].wait_send()

        def issue_s2(rs):
            for p in range(3):
                partner = lax.bitwise_xor(my, ORDERS[p][2])
                for r in rs:
                    d = pltpu.make_async_remote_copy(
                        src_ref=src_of(p, r),
                        dst_ref=bufs[2].at[p].at[
                            pl.ds(r * m_per, m_per), pl.ds(0, PART_COLS[p])],
                        send_sem=ag_ssem.at[p, 2],
                        recv_sem=ag_rsem.at[p, 2],
                        device_id=(partner,), device_id_type=mesh)
                    d.start()
                    descs[(p, 2, r)] = d

        def gemm_s2(rs, scaled):
            for p in range(3):
                for r in rs:
                    oc = ORDERS[p][2] ^ _g(r, ORDERS[p][:2])
                    gemm(p, oc,
                         bufs[2].at[p].at[pl.ds(r * m_per, m_per),
                                          pl.ds(0, PART_COLS[p])],
                         accumulate=True, scaled=scaled and r == rs[-1])

        issue_s2((0, 1))
        for p in range(3):
            for r in range(2):
                oc = ORDERS[p][1] ^ _g(r, ORDERS[p][:1])
                gemm(p, oc,
                     bufs[1].at[p].at[pl.ds(r * m_per, m_per),
                                      pl.ds(0, PART_COLS[p])],
                     accumulate=True)
        for p in range(3):
            for r in (0, 1):
                descs[(p, 2, r)].wait_recv()
        issue_s2((2, 3))
        gemm_s2((0, 1), scaled=False)
        for p in range(3):
            for r in (2, 3):
                descs[(p, 2, r)].wait_recv()
        for p in range(3):
            for r in range(4):
                descs[(p, 2, r)].wait_send()
        gemm_s2((2, 3), scaled=True)

        for desc in a2a:
            desc.wait_send()

        @functools.partial(pl.run_scoped,
                           exit_sem=pltpu.SemaphoreType.REGULAR)
        def _(exit_sem):
            for d in range(1, N_DEV):
                peer = lax.rem(my + d, N_DEV)
                pl.semaphore_signal(exit_sem, inc=1, device_id=(peer,),
                                    device_id_type=mesh)
            pl.semaphore_wait(exit_sem, N_DEV - 1)

    return pl.pallas_call(
        body,
        out_shape=jax.ShapeDtypeStruct((m_per, N), jnp.float32),
        in_specs=[
            pl.BlockSpec(memory_space=pltpu.VMEM),
            pl.BlockSpec(memory_space=pltpu.VMEM),
            pl.BlockSpec(memory_space=pltpu.SMEM),
            pl.BlockSpec(memory_space=pltpu.SMEM),
        ],
        out_specs=pl.BlockSpec(memory_space=pltpu.VMEM),
        scratch_shapes=[
            pltpu.VMEM((m_per, M), jnp.float8_e5m2),
            pltpu.VMEM((3, m_per, max(PART_COLS)), jnp.float8_e5m2),
            pltpu.VMEM((3, 2 * m_per, max(PART_COLS)), jnp.float8_e5m2),
            pltpu.VMEM((3, 4 * m_per, max(PART_COLS)), jnp.float8_e5m2),
            pltpu.SemaphoreType.DMA((N_DEV,)),
            pltpu.SemaphoreType.DMA((N_DEV,)),
            pltpu.SemaphoreType.DMA((3, 3)),
            pltpu.SemaphoreType.DMA((3, 3)),
        ],
        compiler_params=pltpu.CompilerParams(
            collective_id=0, vmem_limit_bytes=64 * 1024 * 1024),
    )(x, w_mat, scale_x, scale_w)


# device time: 176192 ns/iter; 4.0564x vs baseline; 1.0045x over previous
import functools

import jax
import jax.numpy as jnp
from jax import lax
from jax.experimental import pallas as pl
from jax.experimental.pallas import tpu as pltpu

N_DEV = 8
PART_COLS = (2816, 2688, 2688)
PART_OFF = (0, 2816, 5504)
ORDERS = ((1, 3, 4), (3, 4, 1), (4, 1, 3))


def _g(r, masks):
    v = 0
    for j in range(len(masks)):
        if r & (1 << j):
            v ^= masks[j]
    return v


def kernel(x, w_mat, scale_x, scale_w):
    M, _ = x.shape
    _, N = w_mat.shape
    m_per = M // N_DEV

    x = x.astype(jnp.float8_e5m2)
    w_mat = w_mat.astype(jnp.float8_e5m2)

    def body(x_ref, w_ref, sx_ref, sw_ref, out_ref, xr_ref,
             b0_ref, b1_ref, b2_ref,
             a2a_send_sems, a2a_recv_sems, ag_ssem, ag_rsem):
        my = lax.axis_index("i")
        mesh = pl.DeviceIdType.MESH
        bufs = (b0_ref, b1_ref, b2_ref)

        barrier_sem = pltpu.get_barrier_semaphore()
        for d in range(1, N_DEV):
            peer = lax.rem(my + d, N_DEV)
            pl.semaphore_signal(barrier_sem, inc=1, device_id=(peer,),
                                device_id_type=mesh)
        pl.semaphore_wait(barrier_sem, N_DEV - 1)

        a2a = []
        for s in range(1, N_DEV):
            t = lax.bitwise_xor(my, s)
            desc = pltpu.make_async_remote_copy(
                src_ref=x_ref.at[pl.ds(t * m_per, m_per), :],
                dst_ref=xr_ref.at[:, pl.ds(my * m_per, m_per)],
                send_sem=a2a_send_sems.at[s],
                recv_sem=a2a_recv_sems.at[s],
                device_id=(t,), device_id_type=mesh)
            desc.start()
            a2a.append(desc)
        xr_ref[:, pl.ds(my * m_per, m_per)] = x_ref[pl.ds(my * m_per, m_per), :]

        def a2a_wait(slots):
            for s in slots:
                j = lax.bitwise_xor(my, s)
                recv = pltpu.make_async_remote_copy(
                    src_ref=x_ref.at[pl.ds(0, m_per), :],
                    dst_ref=xr_ref.at[:, pl.ds(j * m_per, m_per)],
                    send_sem=a2a_send_sems.at[s],
                    recv_sem=a2a_recv_sems.at[s],
                    device_id=(j,), device_id_type=mesh)
                recv.wait_recv()

        scale = sx_ref[0] * sw_ref[0]

        def src_of(p, r):
            if r == 0:
                return w_ref.at[:, pl.ds(PART_OFF[p], PART_COLS[p])]
            s = r.bit_length() - 1
            return bufs[s].at[p].at[pl.ds((r - (1 << s)) * m_per, m_per),
                                    pl.ds(0, PART_COLS[p])]

        def gemm(p, origin_const, chunk_ref, accumulate, scaled=False):
            o = lax.bitwise_xor(my, origin_const) if origin_const else my
            xo = xr_ref[:, pl.ds(o * m_per, m_per)]
            cp = PART_COLS[p]
            for q in range(2):
                t0 = q * (cp // 2)
                tc = cp // 2
                acc = lax.dot_general(
                    xo, chunk_ref[:, t0:t0 + tc],
                    (((1,), (0,)), ((), ())),
                    preferred_element_type=jnp.float32)
                sl = slice(PART_OFF[p] + t0, PART_OFF[p] + t0 + tc)
                if accumulate:
                    acc = out_ref[:, sl] + acc
                if scaled:
                    acc = acc * scale
                out_ref[:, sl] = acc

        descs = {}
        for t in range(2):
            if t == 1:
                a2a_wait((1, 3, 4))
            for p in range(3):
                mask = ORDERS[p][t]
                partner = lax.bitwise_xor(my, mask)
                for r in range(1 << t):
                    d = pltpu.make_async_remote_copy(
                        src_ref=src_of(p, r),
                        dst_ref=bufs[t].at[p].at[
                            pl.ds(r * m_per, m_per), pl.ds(0, PART_COLS[p])],
                        send_sem=ag_ssem.at[p, t],
                        recv_sem=ag_rsem.at[p, t],
                        device_id=(partner,), device_id_type=mesh)
                    d.start()
                    descs[(p, t, r)] = d

            if t == 0:
                for p in range(3):
                    gemm(p, 0, w_ref.at[:, pl.ds(PART_OFF[p], PART_COLS[p])],
                         accumulate=False)
            else:
                for p in range(3):
                    mask_prev = ORDERS[p][t - 1]
                    for r in range(1 << (t - 1)):
                        oc = mask_prev ^ _g(r, ORDERS[p][:t - 1])
                        gemm(p, oc,
                             bufs[t - 1].at[p].at[pl.ds(r * m_per, m_per),
                                                  pl.ds(0, PART_COLS[p])],
                             accumulate=True)

            for p in range(3):
                for r in range(1 << t):
                    descs[(p, t, r)].wait_recv()
            for p in range(3):
                for r in range(1 << t):
                    descs[(p, t, r)].wait_send()

        def issue_s2(rs):
            for p in range(3):
                partner = lax.bitwise_xor(my, ORDERS[p][2])
                for r in rs:
                    d = pltpu.make_async_remote_copy(
                        src_ref=src_of(p, r),
                        dst_ref=bufs[2].at[p].at[
                            pl.ds(r * m_per, m_per), pl.ds(0, PART_COLS[p])],
                        send_sem=ag_ssem.at[p, 2],
                        recv_sem=ag_rsem.at[p, 2],
                        device_id=(partner,), device_id_type=mesh)
                    d.start()
                    descs[(p, 2, r)] = d

        def gemm_s2(rs, scaled):
            for p in range(3):
                for r in rs:
                    oc = ORDERS[p][2] ^ _g(r, ORDERS[p][:2])
                    gemm(p, oc,
                         bufs[2].at[p].at[pl.ds(r * m_per, m_per),
                                          pl.ds(0, PART_COLS[p])],
                         accumulate=True, scaled=scaled and r == rs[-1])

        a2a_wait((2, 5, 7))
        issue_s2((0, 1))
        for p in range(3):
            for r in range(2):
                oc = ORDERS[p][1] ^ _g(r, ORDERS[p][:1])
                gemm(p, oc,
                     bufs[1].at[p].at[pl.ds(r * m_per, m_per),
                                      pl.ds(0, PART_COLS[p])],
                     accumulate=True)
        for p in range(3):
            for r in (0, 1):
                descs[(p, 2, r)].wait_recv()
        a2a_wait((6,))
        issue_s2((2, 3))
        gemm_s2((0, 1), scaled=False)
        for p in range(3):
            for r in (2, 3):
                descs[(p, 2, r)].wait_recv()
        for p in range(3):
            for r in range(4):
                descs[(p, 2, r)].wait_send()
        gemm_s2((2, 3), scaled=True)

        for desc in a2a:
            desc.wait_send()

        @functools.partial(pl.run_scoped,
                           exit_sem=pltpu.SemaphoreType.REGULAR)
        def _(exit_sem):
            for d in range(1, N_DEV):
                peer = lax.rem(my + d, N_DEV)
                pl.semaphore_signal(exit_sem, inc=1, device_id=(peer,),
                                    device_id_type=mesh)
            pl.semaphore_wait(exit_sem, N_DEV - 1)

    return pl.pallas_call(
        body,
        out_shape=jax.ShapeDtypeStruct((m_per, N), jnp.float32),
        in_specs=[
            pl.BlockSpec(memory_space=pltpu.VMEM),
            pl.BlockSpec(memory_space=pltpu.VMEM),
            pl.BlockSpec(memory_space=pltpu.SMEM),
            pl.BlockSpec(memory_space=pltpu.SMEM),
        ],
        out_specs=pl.BlockSpec(memory_space=pltpu.VMEM),
        scratch_shapes=[
            pltpu.VMEM((m_per, M), jnp.float8_e5m2),
            pltpu.VMEM((3, m_per, max(PART_COLS)), jnp.float8_e5m2),
            pltpu.VMEM((3, 2 * m_per, max(PART_COLS)), jnp.float8_e5m2),
            pltpu.VMEM((3, 4 * m_per, max(PART_COLS)), jnp.float8_e5m2),
            pltpu.SemaphoreType.DMA((N_DEV,)),
            pltpu.SemaphoreType.DMA((N_DEV,)),
            pltpu.SemaphoreType.DMA((3, 3)),
            pltpu.SemaphoreType.DMA((3, 3)),
        ],
        compiler_params=pltpu.CompilerParams(
            collective_id=0, vmem_limit_bytes=64 * 1024 * 1024),
    )(x, w_mat, scale_x, scale_w)
